# baseline (speedup 1.0000x reference)
import functools

import jax
import jax.numpy as jnp
from jax import lax
from jax.experimental import pallas as pl
from jax.experimental.pallas import tpu as pltpu
from jax.experimental.pallas import tpu_sc as plsc

NV = 10000
NE = 160000
D = 512
LANES = 16
NC, NS = 2, 16
NW = NC * NS
CH = 160
CPW = 2
NCHUNK = NW * CPW
NVPAD = NCHUNK * CH
K = 16
BND_PAD = 80


def _sc_spmm(xv, src, dst, vals, bnd):
    mesh = plsc.VectorSubcoreMesh(core_axis_name="c", subcore_axis_name="s")

    @functools.partial(
        pl.kernel,
        mesh=mesh,
        out_type=jax.ShapeDtypeStruct((NVPAD, D), jnp.float32),
        scratch_types=[
            pltpu.VMEM((BND_PAD,), jnp.int32),
            pltpu.VMEM((K,), jnp.int32),
            pltpu.VMEM((K,), jnp.int32),
            pltpu.VMEM((K,), jnp.float32),
            pltpu.VMEM((K, D), jnp.float32),
            pltpu.VMEM((CH, D), jnp.float32),
            pltpu.SemaphoreType.DMA,
        ],
    )
    def body(xv_h, src_h, dst_h, vals_h, bnd_h, out_h,
             bnd_v, idx_v, dst_v, val_v, staged, acc, sem):
        wid = lax.axis_index("s") * NC + lax.axis_index("c")
        pltpu.sync_copy(bnd_h, bnd_v)
        for half in range(CPW):
            c = wid * CPW + half
            base = c * CH
            bv = bnd_v[pl.ds(c, LANES)]
            s = bv[0]
            t = bv[1]
            a0 = (s // K) * K
            nb = (t - a0 + K - 1) // K

            def zero_row(r, carry):
                for cc in range(D // LANES):
                    acc[r, pl.ds(cc * LANES, LANES)] = jnp.zeros(
                        (LANES,), jnp.float32)
                return carry
            lax.fori_loop(0, CH, zero_row, 0)

            def batch(b, carry):
                off = a0 + b * K
                pltpu.sync_copy(src_h.at[pl.ds(off, K)], idx_v)
                pltpu.sync_copy(dst_h.at[pl.ds(off, K)], dst_v)
                pltpu.sync_copy(vals_h.at[pl.ds(off, K)], val_v)
                pltpu.async_copy(xv_h.at[idx_v], staged, sem).wait()
                gvec = off + lax.iota(jnp.int32, LANES)
                okv = jnp.logical_and(gvec >= s, gvec < t)
                vv = jnp.where(okv, val_v[...], 0.0)
                rv = jnp.where(okv, dst_v[...] - base, 0)
                for j in range(K):
                    v = vv[j]
                    r = rv[j]
                    for cc in range(D // LANES):
                        sl = pl.ds(cc * LANES, LANES)
                        acc[r, sl] += v * staged[j, sl]
                return carry
            lax.fori_loop(0, nb, batch, 0)
            pltpu.sync_copy(acc, out_h.at[pl.ds(base, CH)])

    return body(xv, src, dst, vals, bnd)


def _tc_matmul_bias(z, w, b):
    bm = 800

    def mm(z_ref, w_ref, b_ref, o_ref):
        o_ref[...] = jnp.dot(
            z_ref[...], w_ref[...], preferred_element_type=jnp.float32
        ) + b_ref[...]

    return pl.pallas_call(
        mm,
        grid=(z.shape[0] // bm,),
        in_specs=[
            pl.BlockSpec((bm, 128), lambda i: (i, 0)),
            pl.BlockSpec((128, 128), lambda i: (0, 0)),
            pl.BlockSpec((1, 128), lambda i: (0, 0)),
        ],
        out_specs=pl.BlockSpec((bm, 128), lambda i: (i, 0)),
        out_shape=jax.ShapeDtypeStruct((z.shape[0], 128), jnp.float32),
    )(z, w, b[None, :])


def kernel(x, weight, bias, filter_vals, edge_src, edge_dst):
    xv = x.reshape(NV, D)
    src = edge_src.astype(jnp.int32)
    dst = edge_dst.astype(jnp.int32)
    row_starts = jnp.arange(NCHUNK + 1, dtype=jnp.int32) * CH
    bnd = jnp.searchsorted(dst, row_starts, side="left").astype(jnp.int32)
    bnd = jnp.pad(bnd, (0, BND_PAD - (NCHUNK + 1)))
    g = _sc_spmm(xv, src, dst, filter_vals, bnd)
    z = g[:NV].reshape(NV * 4, 128)
    return _tc_matmul_bias(z, weight, bias)


# speedup vs baseline: 1.9439x; 1.9439x over previous
import functools

import jax
import jax.numpy as jnp
from jax import lax
from jax.experimental import pallas as pl
from jax.experimental.pallas import tpu as pltpu
from jax.experimental.pallas import tpu_sc as plsc

NV = 10000
NE = 160000
D = 512
LANES = 16
NC, NS = 2, 16
NW = NC * NS
CH = 160
CPW = 2
NCHUNK = NW * CPW
NVPAD = NCHUNK * CH
K = 32
NEK = NE - K
BND_PAD = 80


def _sc_spmm(xv, meta, vals, bnd):
    mesh = plsc.VectorSubcoreMesh(core_axis_name="c", subcore_axis_name="s")

    @functools.partial(
        pl.kernel,
        mesh=mesh,
        out_type=jax.ShapeDtypeStruct((NVPAD, D), jnp.float32),
        scratch_types=[
            pltpu.VMEM((BND_PAD,), jnp.int32),
            pltpu.VMEM((2, 2 * K), jnp.int32),
            pltpu.VMEM((2, K), jnp.float32),
            pltpu.VMEM((2, K, D), jnp.float32),
            pltpu.VMEM((CH, D), jnp.float32),
            pltpu.SemaphoreType.DMA,
            pltpu.SemaphoreType.DMA,
        ],
    )
    def body(xv_h, meta_h, vals_h, bnd_h, out_h,
             bnd_v, mbuf, vbuf, staged, acc, sem_g, sem_m):
        wid = lax.axis_index("s") * NC + lax.axis_index("c")
        pltpu.sync_copy(bnd_h, bnd_v)

        def offm(a0, b):
            return jnp.minimum(a0 + b * K, NEK)

        def issue_meta(a0, b, buf):
            off = offm(a0, b)
            pltpu.async_copy(meta_h.at[off // K], mbuf.at[buf], sem_m)
            pltpu.async_copy(vals_h.at[pl.ds(off, K)], vbuf.at[buf], sem_m)

        def wait_meta(buf):
            pltpu.make_async_copy(
                meta_h.at[0], mbuf.at[buf], sem_m).wait()
            pltpu.make_async_copy(
                vals_h.at[pl.ds(0, K)], vbuf.at[buf], sem_m).wait()

        def issue_gather(buf):
            pltpu.async_copy(
                xv_h.at[mbuf.at[buf, pl.ds(0, K)]], staged.at[buf], sem_g)

        def wait_gather(buf):
            pltpu.make_async_copy(
                xv_h.at[mbuf.at[buf, pl.ds(0, K)]], staged.at[buf],
                sem_g).wait()

        def run_chunk(half, carry0):
            c = wid * CPW + half
            base = c * CH
            bv = bnd_v[pl.ds(c, LANES)]
            s = bv[0]
            t = bv[1]
            a0 = jnp.minimum((s // K) * K, NEK)
            nb = jnp.maximum((t - a0 + K - 1) // K, 1)

            def zero_row(r, carry):
                for cc in range(D // LANES):
                    acc[r, pl.ds(cc * LANES, LANES)] = jnp.zeros(
                        (LANES,), jnp.float32)
                return carry
            lax.fori_loop(0, CH, zero_row, 0)

            issue_meta(a0, 0, 0)
            wait_meta(0)
            issue_meta(a0, 1, 1)
            issue_gather(0)

            def batch(b, carry):
                cur = jnp.bitwise_and(b, 1)
                nxt = 1 - cur
                off = offm(a0, b)
                wait_gather(cur)
                wait_meta(nxt)
                issue_gather(nxt)
                d0 = mbuf[cur, pl.ds(K, LANES)]
                d1 = mbuf[cur, pl.ds(K + LANES, LANES)]
                w0 = vbuf[cur, pl.ds(0, LANES)]
                w1 = vbuf[cur, pl.ds(LANES, LANES)]
                issue_meta(a0, b + 2, cur)
                g0 = off + lax.iota(jnp.int32, LANES)
                g1 = g0 + LANES
                ok0 = jnp.logical_and(g0 >= s, g0 < t)
                ok1 = jnp.logical_and(g1 >= s, g1 < t)
                vv0 = jnp.where(ok0, w0, 0.0)
                vv1 = jnp.where(ok1, w1, 0.0)
                rv0 = jnp.where(ok0, d0 - base, 0)
                rv1 = jnp.where(ok1, d1 - base, 0)
                for j in range(LANES):
                    v = vv0[j]
                    r = rv0[j]
                    for cc in range(D // LANES):
                        sl = pl.ds(cc * LANES, LANES)
                        plsc.addupdate(acc.at[r, sl], v * staged[cur, j, sl])
                for j in range(LANES):
                    v = vv1[j]
                    r = rv1[j]
                    for cc in range(D // LANES):
                        sl = pl.ds(cc * LANES, LANES)
                        plsc.addupdate(acc.at[r, sl],
                                       v * staged[cur, LANES + j, sl])
                return carry
            lax.fori_loop(0, nb, batch, 0)

            wait_gather(jnp.bitwise_and(nb, 1))
            wait_meta(jnp.bitwise_and(nb + 1, 1))
            pltpu.sync_copy(acc, out_h.at[pl.ds(base, CH)])
            return carry0
        lax.fori_loop(0, CPW, run_chunk, 0)

    return body(xv, meta, vals, bnd)


def _tc_matmul_bias(z, w, b):
    bm = 800

    def mm(z_ref, w_ref, b_ref, o_ref):
        o_ref[...] = jnp.dot(
            z_ref[...], w_ref[...], preferred_element_type=jnp.float32
        ) + b_ref[...]

    return pl.pallas_call(
        mm,
        grid=(z.shape[0] // bm,),
        in_specs=[
            pl.BlockSpec((bm, 128), lambda i: (i, 0)),
            pl.BlockSpec((128, 128), lambda i: (0, 0)),
            pl.BlockSpec((1, 128), lambda i: (0, 0)),
        ],
        out_specs=pl.BlockSpec((bm, 128), lambda i: (i, 0)),
        out_shape=jax.ShapeDtypeStruct((z.shape[0], 128), jnp.float32),
    )(z, w, b[None, :])


def kernel(x, weight, bias, filter_vals, edge_src, edge_dst):
    xv = x.reshape(NV, D)
    src = edge_src.astype(jnp.int32)
    dst = edge_dst.astype(jnp.int32)
    meta = jnp.concatenate(
        [src.reshape(NE // K, K), dst.reshape(NE // K, K)], axis=1)
    row_starts = jnp.arange(NCHUNK + 1, dtype=jnp.int32) * CH
    bnd = jnp.searchsorted(dst, row_starts, side="left").astype(jnp.int32)
    bnd = jnp.pad(bnd, (0, BND_PAD - (NCHUNK + 1)))
    g = _sc_spmm(xv, meta, filter_vals, bnd)
    z = g[:NV].reshape(NV * 4, 128)
    return _tc_matmul_bias(z, weight, bias)


# speedup vs baseline: 3.0883x; 1.5887x over previous
import functools

import jax
import jax.numpy as jnp
from jax import lax
from jax.experimental import pallas as pl
from jax.experimental.pallas import tpu as pltpu
from jax.experimental.pallas import tpu_sc as plsc

NV = 10000
NE = 160000
D = 512
LANES = 16
NSTEP = D // LANES
NC, NS = 2, 16
NW = NC * NS
CH = 80
CPW = 4
NCHUNK = NW * CPW
NVPAD = NCHUNK * CH
K = 32
NEK = NE - K
BND_PAD = 144


def _sc_spmm(xv, meta, vals, bnd):
    mesh = plsc.VectorSubcoreMesh(core_axis_name="c", subcore_axis_name="s")

    @functools.partial(
        pl.kernel,
        mesh=mesh,
        out_type=jax.ShapeDtypeStruct((NVPAD, D), jnp.float32),
        scratch_types=[
            pltpu.VMEM((BND_PAD,), jnp.int32),
            pltpu.VMEM((2, 2 * K), jnp.int32),
            pltpu.VMEM((2, K), jnp.float32),
            pltpu.VMEM((4, K, D), jnp.float32),
            pltpu.VMEM((CH, D), jnp.float32),
            pltpu.VMEM((D,), jnp.float32),
            pltpu.SemaphoreType.DMA,
            pltpu.SemaphoreType.DMA,
        ],
    )
    def body(xv_h, meta_h, vals_h, bnd_h, out_h,
             bnd_v, mbuf, vbuf, staged, acc, regbuf, sem_g, sem_m):
        wid = lax.axis_index("s") * NC + lax.axis_index("c")
        pltpu.sync_copy(bnd_h, bnd_v)

        def offm(a0, b):
            return jnp.minimum(a0 + b * K, NEK)

        def issue_meta(a0, b, buf):
            off = offm(a0, b)
            pltpu.async_copy(meta_h.at[off // K], mbuf.at[buf], sem_m)
            pltpu.async_copy(vals_h.at[pl.ds(off, K)], vbuf.at[buf], sem_m)

        def wait_meta(buf):
            pltpu.make_async_copy(
                meta_h.at[0], mbuf.at[buf], sem_m).wait()
            pltpu.make_async_copy(
                vals_h.at[pl.ds(0, K)], vbuf.at[buf], sem_m).wait()

        def issue_gather(mb, buf):
            pltpu.async_copy(
                xv_h.at[mbuf.at[mb, pl.ds(0, K)]], staged.at[buf], sem_g)

        def wait_gather():
            pltpu.make_async_copy(
                xv_h.at[mbuf.at[0, pl.ds(0, K)]], staged.at[0],
                sem_g).wait()

        zero16 = jnp.zeros((LANES,), jnp.float32)

        def run_chunk(half, carry0):
            c = wid * CPW + half
            base = c * CH
            bv = bnd_v[pl.ds(c, LANES)]
            s = bv[0]
            t = bv[1]
            a0 = jnp.minimum((s // K) * K, NEK)
            nb = jnp.maximum((t - a0 + K - 1) // K, 1)

            def zero_row(r, carry):
                for cc in range(NSTEP):
                    acc[r, pl.ds(cc * LANES, LANES)] = zero16
                return carry
            lax.fori_loop(0, CH, zero_row, 0)

            issue_meta(a0, 0, 0)
            wait_meta(0)
            issue_meta(a0, 1, 1)
            issue_gather(0, 0)

            def batch(b, prev_r):
                regacc = tuple(
                    regbuf[pl.ds(cc * LANES, LANES)] for cc in range(NSTEP))
                cur = jnp.bitwise_and(b, 3)
                nxt = jnp.bitwise_and(b + 1, 3)
                curm = jnp.bitwise_and(b, 1)
                nxtm = 1 - curm
                off = offm(a0, b)
                wait_gather()
                wait_meta(nxtm)
                issue_gather(nxtm, nxt)
                d0 = mbuf[curm, pl.ds(K, LANES)]
                d1 = mbuf[curm, pl.ds(K + LANES, LANES)]
                w0 = vbuf[curm, pl.ds(0, LANES)]
                w1 = vbuf[curm, pl.ds(LANES, LANES)]
                issue_meta(a0, b + 2, curm)
                g0 = off + lax.iota(jnp.int32, LANES)
                g1 = g0 + LANES
                ok0 = jnp.logical_and(g0 >= s, g0 < t)
                ok1 = jnp.logical_and(g1 >= s, g1 < t)
                vv0 = jnp.where(ok0, w0, 0.0)
                vv1 = jnp.where(ok1, w1, 0.0)
                rv0 = jnp.where(ok0, d0 - base, 0)
                rv1 = jnp.where(ok1, d1 - base, 0)

                def edge(jrow, v, prev_r, regacc):
                    r = jrow[0]
                    j = jrow[1]
                    changed = r != prev_r

                    @pl.when(changed)
                    def flush():
                        for cc in range(NSTEP):
                            plsc.addupdate(
                                acc.at[prev_r, pl.ds(cc * LANES, LANES)],
                                regacc[cc])

                    keep = jnp.where(changed, 0.0, 1.0)
                    regacc = tuple(
                        regacc[cc] * keep
                        + v * staged[cur, j, pl.ds(cc * LANES, LANES)]
                        for cc in range(NSTEP))
                    return r, regacc

                for j in range(LANES):
                    prev_r, regacc = edge(
                        (rv0[j], j), vv0[j], prev_r, regacc)
                for j in range(LANES):
                    prev_r, regacc = edge(
                        (rv1[j], LANES + j), vv1[j], prev_r, regacc)
                for cc in range(NSTEP):
                    regbuf[pl.ds(cc * LANES, LANES)] = regacc[cc]
                return prev_r

            for cc in range(NSTEP):
                regbuf[pl.ds(cc * LANES, LANES)] = zero16
            last_r = lax.fori_loop(0, nb, batch, jnp.int32(0))
            for cc in range(NSTEP):
                sl = pl.ds(cc * LANES, LANES)
                plsc.addupdate(acc.at[last_r, sl], regbuf[sl])

            wait_gather()
            wait_meta(jnp.bitwise_and(nb + 1, 1))
            pltpu.sync_copy(acc, out_h.at[pl.ds(base, CH)])
            return carry0
        lax.fori_loop(0, CPW, run_chunk, 0)

    return body(xv, meta, vals, bnd)


def _tc_matmul_bias(z, w, b):
    bm = 800

    def mm(z_ref, w_ref, b_ref, o_ref):
        o_ref[...] = jnp.dot(
            z_ref[...], w_ref[...], preferred_element_type=jnp.float32
        ) + b_ref[...]

    return pl.pallas_call(
        mm,
        grid=(z.shape[0] // bm,),
        in_specs=[
            pl.BlockSpec((bm, 128), lambda i: (i, 0)),
            pl.BlockSpec((128, 128), lambda i: (0, 0)),
            pl.BlockSpec((1, 128), lambda i: (0, 0)),
        ],
        out_specs=pl.BlockSpec((bm, 128), lambda i: (i, 0)),
        out_shape=jax.ShapeDtypeStruct((z.shape[0], 128), jnp.float32),
    )(z, w, b[None, :])


def kernel(x, weight, bias, filter_vals, edge_src, edge_dst):
    xv = x.reshape(NV, D)
    src = edge_src.astype(jnp.int32)
    dst = edge_dst.astype(jnp.int32)
    meta = jnp.concatenate(
        [src.reshape(NE // K, K), dst.reshape(NE // K, K)], axis=1)
    row_starts = jnp.arange(NCHUNK + 1, dtype=jnp.int32) * CH
    bnd = jnp.searchsorted(dst, row_starts, side="left").astype(jnp.int32)
    bnd = jnp.pad(bnd, (0, BND_PAD - (NCHUNK + 1)))
    g = _sc_spmm(xv, meta, filter_vals, bnd)
    z = g[:NV].reshape(NV * 4, 128)
    return _tc_matmul_bias(z, weight, bias)


# speedup vs baseline: 3.2708x; 1.0591x over previous
import functools

import jax
import jax.numpy as jnp
from jax import lax
from jax.experimental import pallas as pl
from jax.experimental.pallas import tpu as pltpu
from jax.experimental.pallas import tpu_sc as plsc

NV = 10000
NE = 160000
D = 512
LANES = 16
NSTEP = D // LANES
NC, NS = 2, 16
NW = NC * NS
CH = 80
CPW = 4
NCHUNK = NW * CPW
NVPAD = NCHUNK * CH
K = 32
NEK = NE - K
BND_PAD = 144


def _sc_spmm(xv, meta, vals, bnd):
    mesh = plsc.VectorSubcoreMesh(core_axis_name="c", subcore_axis_name="s")

    @functools.partial(
        pl.kernel,
        mesh=mesh,
        out_type=jax.ShapeDtypeStruct((NVPAD, D), jnp.float32),
        scratch_types=[
            pltpu.VMEM((BND_PAD,), jnp.int32),
            pltpu.VMEM((2, 2 * K), jnp.int32),
            pltpu.VMEM((2, K), jnp.float32),
            pltpu.VMEM((4, K, D), jnp.float32),
            pltpu.VMEM((CH + 1, D), jnp.float32),
            pltpu.VMEM((D,), jnp.float32),
            pltpu.SemaphoreType.DMA,
            pltpu.SemaphoreType.DMA,
        ],
    )
    def body(xv_h, meta_h, vals_h, bnd_h, out_h,
             bnd_v, mbuf, vbuf, staged, acc, regbuf, sem_g, sem_m):
        wid = lax.axis_index("s") * NC + lax.axis_index("c")
        pltpu.sync_copy(bnd_h, bnd_v)

        def offm(a0, b):
            return jnp.minimum(a0 + b * K, NEK)

        def issue_meta(a0, b, buf):
            off = offm(a0, b)
            pltpu.async_copy(meta_h.at[off // K], mbuf.at[buf], sem_m)
            pltpu.async_copy(vals_h.at[pl.ds(off, K)], vbuf.at[buf], sem_m)

        def wait_meta(buf):
            pltpu.make_async_copy(
                meta_h.at[0], mbuf.at[buf], sem_m).wait()
            pltpu.make_async_copy(
                vals_h.at[pl.ds(0, K)], vbuf.at[buf], sem_m).wait()

        def issue_gather(mb, buf):
            pltpu.async_copy(
                xv_h.at[mbuf.at[mb, pl.ds(0, K)]], staged.at[buf], sem_g)

        def wait_gather():
            pltpu.make_async_copy(
                xv_h.at[mbuf.at[0, pl.ds(0, K)]], staged.at[0],
                sem_g).wait()

        zero16 = jnp.zeros((LANES,), jnp.float32)

        def run_chunk(half, carry0):
            c = wid * CPW + half
            base = c * CH
            bv = bnd_v[pl.ds(c, LANES)]
            s = bv[0]
            t = bv[1]
            a0 = jnp.minimum((s // K) * K, NEK)
            nb = jnp.maximum((t - a0 + K - 1) // K, 1)

            def zero_row(r, carry):
                for cc in range(NSTEP):
                    acc[r, pl.ds(cc * LANES, LANES)] = zero16
                return carry
            lax.fori_loop(0, CH, zero_row, 0)

            issue_meta(a0, 0, 0)
            wait_meta(0)
            issue_meta(a0, 1, 1)
            issue_gather(0, 0)

            def batch(b, prev_r):
                regacc = tuple(
                    regbuf[pl.ds(cc * LANES, LANES)] for cc in range(NSTEP))
                cur = jnp.bitwise_and(b, 3)
                nxt = jnp.bitwise_and(b + 1, 3)
                curm = jnp.bitwise_and(b, 1)
                nxtm = 1 - curm
                off = offm(a0, b)
                wait_gather()
                wait_meta(nxtm)
                issue_gather(nxtm, nxt)
                d0 = mbuf[curm, pl.ds(K, LANES)]
                d1 = mbuf[curm, pl.ds(K + LANES, LANES)]
                w0 = vbuf[curm, pl.ds(0, LANES)]
                w1 = vbuf[curm, pl.ds(LANES, LANES)]
                issue_meta(a0, b + 2, curm)
                g0 = off + lax.iota(jnp.int32, LANES)
                g1 = g0 + LANES
                ok0 = jnp.logical_and(g0 >= s, g0 < t)
                ok1 = jnp.logical_and(g1 >= s, g1 < t)
                vv0 = jnp.where(ok0, w0, 0.0)
                vv1 = jnp.where(ok1, w1, 0.0)
                rv0 = jnp.where(ok0, d0 - base, CH)
                rv1 = jnp.where(ok1, d1 - base, CH)

                def edge(jrow, v, prev_r, regacc):
                    r = jrow[0]
                    j = jrow[1]
                    keep = jnp.where(r != prev_r, 0.0, 1.0)
                    regacc = tuple(
                        regacc[cc] * keep
                        + v * staged[cur, j, pl.ds(cc * LANES, LANES)]
                        for cc in range(NSTEP))
                    for cc in range(NSTEP):
                        acc[r, pl.ds(cc * LANES, LANES)] = regacc[cc]
                    return r, regacc

                for j in range(LANES):
                    prev_r, regacc = edge(
                        (rv0[j], j), vv0[j], prev_r, regacc)
                for j in range(LANES):
                    prev_r, regacc = edge(
                        (rv1[j], LANES + j), vv1[j], prev_r, regacc)
                for cc in range(NSTEP):
                    regbuf[pl.ds(cc * LANES, LANES)] = regacc[cc]
                return prev_r

            for cc in range(NSTEP):
                regbuf[pl.ds(cc * LANES, LANES)] = zero16
            lax.fori_loop(0, nb, batch, jnp.int32(CH))

            wait_gather()
            wait_meta(jnp.bitwise_and(nb + 1, 1))
            pltpu.sync_copy(acc.at[pl.ds(0, CH)], out_h.at[pl.ds(base, CH)])
            return carry0
        lax.fori_loop(0, CPW, run_chunk, 0)

    return body(xv, meta, vals, bnd)


def _tc_matmul_bias(z, w, b):
    bm = 800

    def mm(z_ref, w_ref, b_ref, o_ref):
        o_ref[...] = jnp.dot(
            z_ref[...], w_ref[...], preferred_element_type=jnp.float32
        ) + b_ref[...]

    return pl.pallas_call(
        mm,
        grid=(z.shape[0] // bm,),
        in_specs=[
            pl.BlockSpec((bm, 128), lambda i: (i, 0)),
            pl.BlockSpec((128, 128), lambda i: (0, 0)),
            pl.BlockSpec((1, 128), lambda i: (0, 0)),
        ],
        out_specs=pl.BlockSpec((bm, 128), lambda i: (i, 0)),
        out_shape=jax.ShapeDtypeStruct((z.shape[0], 128), jnp.float32),
    )(z, w, b[None, :])


def kernel(x, weight, bias, filter_vals, edge_src, edge_dst):
    xv = x.reshape(NV, D)
    src = edge_src.astype(jnp.int32)
    dst = edge_dst.astype(jnp.int32)
    meta = jnp.concatenate(
        [src.reshape(NE // K, K), dst.reshape(NE // K, K)], axis=1)
    row_starts = jnp.arange(NCHUNK + 1, dtype=jnp.int32) * CH
    bnd = jnp.searchsorted(dst, row_starts, side="left").astype(jnp.int32)
    bnd = jnp.pad(bnd, (0, BND_PAD - (NCHUNK + 1)))
    g = _sc_spmm(xv, meta, filter_vals, bnd)
    z = g[:NV].reshape(NV * 4, 128)
    return _tc_matmul_bias(z, weight, bias)
